# baseline (speedup 1.0000x reference)
import functools

import jax
import jax.numpy as jnp
from jax import lax
from jax.experimental import pallas as pl
from jax.experimental.pallas import tpu as pltpu
from jax.experimental.pallas import tpu_sc as plsc

NUM_BEV = 64
NX = 432
NY = 496
P = 30000
NCOLS = NX * NY
NW = 32
RANGE = NCOLS // NW
MAP_PAD = 6720
BLK = 512
NFULL = RANGE // BLK
REM = RANGE - NFULL * BLK
CHUNK = 128
NVEC = P // 16
LISTSZ = BLK + CHUNK


def _make_kernel():
    mesh = plsc.VectorSubcoreMesh(core_axis_name="c", subcore_axis_name="s")

    @functools.partial(
        pl.kernel,
        mesh=mesh,
        compiler_params=pltpu.CompilerParams(needs_layout_passes=False),
        out_type=jax.ShapeDtypeStruct((NUM_BEV * NCOLS,), jnp.float32),
        scratch_types=[
            pltpu.VMEM((P,), jnp.int32),
            pltpu.VMEM((MAP_PAD,), jnp.int32),
            pltpu.VMEM((NUM_BEV * BLK,), jnp.float32),
            pltpu.VMEM((CHUNK, 2 * NUM_BEV), jnp.float32),
            pltpu.VMEM((CHUNK, 2 * NUM_BEV), jnp.float32),
            pltpu.VMEM((2 * LISTSZ,), jnp.int32),
            pltpu.VMEM((2 * LISTSZ,), jnp.int32),
            pltpu.VMEM((2 * LISTSZ,), jnp.int32),
            pltpu.SemaphoreType.DMA,
            pltpu.SemaphoreType.DMA,
            pltpu.SemaphoreType.DMA,
        ],
    )
    def scatter_kernel(idx_hbm, feat_hbm, out_hbm, idx_w, map_v, out_b,
                       rows_a, rows_b, plist, hlist, poslist, sem_a, sem_b,
                       sem_out):
        j16 = lax.iota(jnp.int32, 16)
        zeros16f = jnp.zeros((16,), jnp.float32)
        wid = lax.axis_index("s") * 2 + lax.axis_index("c")
        base = wid * RANGE

        idx_cp = pltpu.async_copy(idx_hbm, idx_w, sem_b)

        def init_map(v, _):
            for u in range(4):
                map_v[pl.ds(v * 64 + u * 16, 16)] = jnp.full(
                    (16,), -1, jnp.int32)
            return 0

        lax.fori_loop(0, MAP_PAD // 64, init_map, 0)

        def init_out(v, _):
            for u in range(8):
                out_b[pl.ds(v * 128 + u * 16, 16)] = zeros16f
            return 0

        lax.fori_loop(0, (NUM_BEV * BLK) // 128, init_out, 0)

        idx_cp.wait()

        def vec_body(v, _):
            for u in range(5):
                idxv = idx_w[pl.ds(v * 80 + u * 16, 16)]
                local = idxv - base
                inr = (local >= 0) & (local < RANGE)
                pv = j16 + v * 80 + u * 16
                safe = jnp.where(inr, local, jnp.zeros((16,), jnp.int32))
                plsc.store_scatter(map_v, [safe], pv, mask=inr)
            return 0

        lax.fori_loop(0, NVEC // 5, vec_body, 0)

        def scan_block(bcol, off, nscan):
            cnts = []
            for v in range(nscan):
                m16 = map_v[pl.ds(bcol + v * 16, 16)]
                cnts.append(jnp.sum((m16 >= 0).astype(jnp.int32)))
            offs = [jnp.int32(0)]
            for v in range(nscan):
                offs.append(offs[v] + cnts[v])
            for v in range(nscan):
                m16 = map_v[pl.ds(bcol + v * 16, 16)]
                wr = m16 >= 0
                o = off + offs[v]
                plsc.store_compressed(plist.at[pl.ds(o, 16)], m16, mask=wr)
                plsc.store_compressed(hlist.at[pl.ds(o, 16)],
                                      lax.shift_right_logical(m16, 1),
                                      mask=wr)
                plsc.store_compressed(poslist.at[pl.ds(o, 16)],
                                      j16 + v * 16, mask=wr)
            n = offs[nscan]
            for k in range(CHUNK // 16):
                hlist[pl.ds(off + n + k * 16, 16)] = j16 + k * 16
                plist[pl.ds(off + n + k * 16, 16)] = j16 + k * 16
                poslist[pl.ds(off + n + k * 16, 16)] = j16
            return n

        def rezero(off, n_prev):
            def z_body(v, _):
                posg = poslist[pl.ds(off + v * 16, 16)]
                zvalid = (v * 16 + j16) < n_prev

                @plsc.parallel_loop(0, NUM_BEV, step=1, unroll=16)
                def _(ch):
                    plsc.store_scatter(out_b, [posg + ch * BLK], zeros16f,
                                       mask=zvalid)
                return 0

            lax.fori_loop(0, (n_prev + 15) // 16, z_body, 0)

        def fire_first(off, n):
            nch = (n + (CHUNK - 1)) // CHUNK

            @pl.when(nch > 0)
            def _():
                idx_ref = hlist.at[pl.ds(off, CHUNK)]
                pltpu.async_copy(feat_hbm.at[idx_ref], rows_a, sem_a)

        def transpose_block(off, n):
            nch = (n + (CHUNK - 1)) // CHUNK

            def fire(c, buf, sem):
                idx_ref = hlist.at[pl.ds(off + c * CHUNK, CHUNK)]
                pltpu.async_copy(feat_hbm.at[idx_ref], buf, sem)

            def wait_chunk(buf, sem):
                pltpu.make_async_copy(feat_hbm.at[pl.ds(0, CHUNK)], buf,
                                      sem).wait()

            def do_chunk(c, buf):
                for g in range(CHUNK // 16):
                    rbase = c * CHUNK + g * 16
                    posg = poslist[pl.ds(off + rbase, 16)]
                    pg = plist[pl.ds(off + rbase, 16)]
                    parbase = (pg & 1) * NUM_BEV
                    rowvalid = (rbase + j16) < n
                    rowsel = j16 + g * 16

                    @plsc.parallel_loop(0, NUM_BEV, step=1, unroll=16)
                    def _(ch):
                        vals = plsc.load_gather(buf, [rowsel, parbase + ch])
                        plsc.store_scatter(out_b, [posg + ch * BLK],
                                           vals, mask=rowvalid)

            def pair_body(q, _):
                c0 = 2 * q
                c1 = 2 * q + 1

                @pl.when(c1 < nch)
                def _():
                    fire(c1, rows_b, sem_b)

                wait_chunk(rows_a, sem_a)
                do_chunk(c0, rows_a)

                @pl.when(c1 < nch)
                def _():
                    @pl.when(c1 + 1 < nch)
                    def _():
                        fire(c1 + 1, rows_a, sem_a)

                    wait_chunk(rows_b, sem_b)
                    do_chunk(c1, rows_b)

                return 0

            lax.fori_loop(0, (nch + 1) // 2, pair_body, 0)

        def fire_out(bcol, width):
            for ch in range(NUM_BEV):
                pltpu.async_copy(
                    out_b.at[pl.ds(ch * BLK, width)],
                    out_hbm.at[pl.ds(ch * NCOLS + base + bcol, width)],
                    sem_out)

        def drain_out(nwords):
            pltpu.make_async_copy(
                out_hbm.at[pl.ds(0, nwords)],
                out_b.at[pl.ds(0, nwords)],
                sem_out).wait()

        n0 = scan_block(jnp.int32(0), jnp.int32(0), BLK // 16)
        fire_first(jnp.int32(0), n0)
        transpose_block(jnp.int32(0), n0)
        fire_out(jnp.int32(0), BLK)

        def block_body(b, n_prev):
            off = (b % 2) * LISTSZ
            prev_off = ((b + 1) % 2) * LISTSZ
            n = scan_block(b * BLK, off, BLK // 16)
            fire_first(off, n)
            drain_out(NUM_BEV * BLK)
            rezero(prev_off, n_prev)
            transpose_block(off, n)
            fire_out(b * BLK, BLK)
            return n

        n_last = lax.fori_loop(1, NFULL, block_body, n0)

        off = (NFULL % 2) * LISTSZ
        prev_off = ((NFULL + 1) % 2) * LISTSZ
        n = scan_block(jnp.int32(NFULL * BLK), off, (REM + 15) // 16)
        fire_first(off, n)
        drain_out(NUM_BEV * BLK)
        rezero(prev_off, n_last)
        transpose_block(off, n)
        fire_out(jnp.int32(NFULL * BLK), REM)
        drain_out(NUM_BEV * REM)

    return scatter_kernel


_scatter = _make_kernel()


@jax.jit
def kernel(pillar_features, voxel_coords):
    idx = (voxel_coords[:, 0] + voxel_coords[:, 1]
           + voxel_coords[:, 2] * NY).astype(jnp.int32)
    feat_pairs = pillar_features.reshape(P // 2, 2 * NUM_BEV)
    out = _scatter(idx, feat_pairs)
    return out.reshape(1, NUM_BEV, NX, NY)


# speedup vs baseline: 1.1034x; 1.1034x over previous
import functools

import jax
import jax.numpy as jnp
from jax import lax
from jax.experimental import pallas as pl
from jax.experimental.pallas import tpu as pltpu
from jax.experimental.pallas import tpu_sc as plsc

NUM_BEV = 64
NX = 432
NY = 496
P = 30000
NCOLS = NX * NY
NW = 32
RANGE = NCOLS // NW
MAP_PAD = 6720
BLK = 512
NFULL = RANGE // BLK
REM = RANGE - NFULL * BLK
CHUNK = 64
NVEC = P // 16
LISTSZ = BLK + CHUNK


def _make_kernel():
    mesh = plsc.VectorSubcoreMesh(core_axis_name="c", subcore_axis_name="s")

    @functools.partial(
        pl.kernel,
        mesh=mesh,
        compiler_params=pltpu.CompilerParams(needs_layout_passes=False),
        out_type=jax.ShapeDtypeStruct((NUM_BEV * NCOLS,), jnp.float32),
        scratch_types=[
            pltpu.VMEM((P,), jnp.int32),
            pltpu.VMEM((MAP_PAD,), jnp.int32),
            pltpu.VMEM((NUM_BEV * BLK,), jnp.float32),
            pltpu.VMEM((CHUNK, 2 * NUM_BEV), jnp.float32),
            pltpu.VMEM((CHUNK, 2 * NUM_BEV), jnp.float32),
            pltpu.VMEM((2 * LISTSZ,), jnp.int32),
            pltpu.VMEM((2 * LISTSZ,), jnp.int32),
            pltpu.VMEM((2 * LISTSZ,), jnp.int32),
            pltpu.SemaphoreType.DMA,
            pltpu.SemaphoreType.DMA,
            pltpu.SemaphoreType.DMA,
        ],
    )
    def scatter_kernel(idx_hbm, feat_hbm, out_hbm, idx_w, map_v, out_b,
                       rows_a, rows_b, plist, hlist, poslist, sem_a, sem_b,
                       sem_out):
        j16 = lax.iota(jnp.int32, 16)
        zeros16f = jnp.zeros((16,), jnp.float32)
        wid = lax.axis_index("s") * 2 + lax.axis_index("c")
        base = wid * RANGE

        idx_cp = pltpu.async_copy(idx_hbm, idx_w, sem_b)

        def init_map(v, _):
            for u in range(4):
                map_v[pl.ds(v * 64 + u * 16, 16)] = jnp.full(
                    (16,), -1, jnp.int32)
            return 0

        lax.fori_loop(0, MAP_PAD // 64, init_map, 0)

        def init_out(v, _):
            for u in range(8):
                out_b[pl.ds(v * 128 + u * 16, 16)] = zeros16f
            return 0

        lax.fori_loop(0, (NUM_BEV * BLK) // 128, init_out, 0)

        idx_cp.wait()

        def vec_body(v, _):
            for u in range(5):
                idxv = idx_w[pl.ds(v * 80 + u * 16, 16)]
                local = idxv - base
                inr = (local >= 0) & (local < RANGE)
                pv = j16 + v * 80 + u * 16
                safe = jnp.where(inr, local, jnp.zeros((16,), jnp.int32))
                plsc.store_scatter(map_v, [safe], pv, mask=inr)
            return 0

        lax.fori_loop(0, NVEC // 5, vec_body, 0)

        def scan_block(bcol, off, nscan):
            cnts = []
            for v in range(nscan):
                m16 = map_v[pl.ds(bcol + v * 16, 16)]
                cnts.append(jnp.sum((m16 >= 0).astype(jnp.int32)))
            offs = [jnp.int32(0)]
            for v in range(nscan):
                offs.append(offs[v] + cnts[v])
            for v in range(nscan):
                m16 = map_v[pl.ds(bcol + v * 16, 16)]
                wr = m16 >= 0
                o = off + offs[v]
                plsc.store_compressed(plist.at[pl.ds(o, 16)], m16, mask=wr)
                plsc.store_compressed(hlist.at[pl.ds(o, 16)],
                                      lax.shift_right_logical(m16, 1),
                                      mask=wr)
                plsc.store_compressed(poslist.at[pl.ds(o, 16)],
                                      j16 + v * 16, mask=wr)
            n = offs[nscan]
            for k in range(CHUNK // 16):
                hlist[pl.ds(off + n + k * 16, 16)] = j16 + k * 16
                plist[pl.ds(off + n + k * 16, 16)] = j16 + k * 16
                poslist[pl.ds(off + n + k * 16, 16)] = j16
            return n

        def rezero(off, n_prev):
            def z_body(v, _):
                posg = poslist[pl.ds(off + v * 16, 16)]
                zvalid = (v * 16 + j16) < n_prev

                @plsc.parallel_loop(0, NUM_BEV, step=1, unroll=16)
                def _(ch):
                    plsc.store_scatter(out_b, [posg + ch * BLK], zeros16f,
                                       mask=zvalid)
                return 0

            lax.fori_loop(0, (n_prev + 15) // 16, z_body, 0)

        def fire_first(off, n):
            nch = (n + (CHUNK - 1)) // CHUNK

            @pl.when(nch > 0)
            def _():
                idx_ref = hlist.at[pl.ds(off, CHUNK)]
                pltpu.async_copy(feat_hbm.at[idx_ref], rows_a, sem_a)

        def transpose_block(off, n):
            nch = (n + (CHUNK - 1)) // CHUNK

            def fire(c, buf, sem):
                idx_ref = hlist.at[pl.ds(off + c * CHUNK, CHUNK)]
                pltpu.async_copy(feat_hbm.at[idx_ref], buf, sem)

            def wait_chunk(buf, sem):
                pltpu.make_async_copy(feat_hbm.at[pl.ds(0, CHUNK)], buf,
                                      sem).wait()

            def do_chunk(c, buf):
                for g in range(CHUNK // 16):
                    rbase = c * CHUNK + g * 16
                    posg = poslist[pl.ds(off + rbase, 16)]
                    pg = plist[pl.ds(off + rbase, 16)]
                    parbase = (pg & 1) * NUM_BEV
                    rowvalid = (rbase + j16) < n
                    rowsel = j16 + g * 16

                    @plsc.parallel_loop(0, NUM_BEV, step=1, unroll=16)
                    def _(ch):
                        vals = plsc.load_gather(buf, [rowsel, parbase + ch])
                        plsc.store_scatter(out_b, [posg + ch * BLK],
                                           vals, mask=rowvalid)

            def pair_body(q, _):
                c0 = 2 * q
                c1 = 2 * q + 1

                @pl.when(c1 < nch)
                def _():
                    fire(c1, rows_b, sem_b)

                wait_chunk(rows_a, sem_a)
                do_chunk(c0, rows_a)

                @pl.when(c1 < nch)
                def _():
                    @pl.when(c1 + 1 < nch)
                    def _():
                        fire(c1 + 1, rows_a, sem_a)

                    wait_chunk(rows_b, sem_b)
                    do_chunk(c1, rows_b)

                return 0

            lax.fori_loop(0, (nch + 1) // 2, pair_body, 0)

        def fire_out(bcol, width):
            for ch in range(NUM_BEV):
                pltpu.async_copy(
                    out_b.at[pl.ds(ch * BLK, width)],
                    out_hbm.at[pl.ds(ch * NCOLS + base + bcol, width)],
                    sem_out)

        def drain_out(nwords):
            pltpu.make_async_copy(
                out_hbm.at[pl.ds(0, nwords)],
                out_b.at[pl.ds(0, nwords)],
                sem_out).wait()

        n0 = scan_block(jnp.int32(0), jnp.int32(0), BLK // 16)
        fire_first(jnp.int32(0), n0)
        transpose_block(jnp.int32(0), n0)
        fire_out(jnp.int32(0), BLK)

        def block_body(b, n_prev):
            off = (b % 2) * LISTSZ
            prev_off = ((b + 1) % 2) * LISTSZ
            n = scan_block(b * BLK, off, BLK // 16)
            fire_first(off, n)
            drain_out(NUM_BEV * BLK)
            rezero(prev_off, n_prev)
            transpose_block(off, n)
            fire_out(b * BLK, BLK)
            return n

        n_last = lax.fori_loop(1, NFULL, block_body, n0)

        off = (NFULL % 2) * LISTSZ
        prev_off = ((NFULL + 1) % 2) * LISTSZ
        n = scan_block(jnp.int32(NFULL * BLK), off, (REM + 15) // 16)
        fire_first(off, n)
        drain_out(NUM_BEV * BLK)
        rezero(prev_off, n_last)
        transpose_block(off, n)
        fire_out(jnp.int32(NFULL * BLK), REM)
        drain_out(NUM_BEV * REM)

    return scatter_kernel


_scatter = _make_kernel()


@jax.jit
def kernel(pillar_features, voxel_coords):
    idx = (voxel_coords[:, 0] + voxel_coords[:, 1]
           + voxel_coords[:, 2] * NY).astype(jnp.int32)
    feat_pairs = pillar_features.reshape(P // 2, 2 * NUM_BEV)
    out = _scatter(idx, feat_pairs)
    return out.reshape(1, NUM_BEV, NX, NY)
